# Initial kernel scaffold; baseline (speedup 1.0000x reference)
import jax
import jax.numpy as jnp
from jax.experimental import pallas as pl


def kernel(feat0, feat1, feat2, feat3, boxes):
    raise NotImplementedError("write your pallas kernel here")


# speedup vs baseline: 9.1669x; 9.1669x over previous
import functools

import jax
import jax.numpy as jnp
from jax import lax
from jax.experimental import pallas as pl
from jax.experimental.pallas import tpu as pltpu
from jax.experimental.pallas import tpu_sc as plsc

SCALES = (0.25, 0.125, 0.0625, 0.03125)
SIZES = (256, 128, 64, 32)
BASES = (0, 65536, 81920, 86016)
C = 256
OUT_H = 7
OUT_W = 7
NBINS = OUT_H * OUT_W
NCON = 16
NC = 2
NS = 16
NW = NC * NS
G = 8


def _iw_kernel(bx_ref, lv_ref, idx_ref, wt_ref):
    b = pl.program_id(0)
    ph = b // OUT_W
    pw = b - ph * OUT_W
    x1 = bx_ref[0]
    y1 = bx_ref[1]
    x2 = bx_ref[2]
    y2 = bx_ref[3]
    lv = lv_ref[...]

    def sel(vals, dtype):
        out = jnp.full(lv.shape, vals[3], dtype)
        for l in (2, 1, 0):
            out = jnp.where(lv == l, jnp.asarray(vals[l], dtype), out)
        return out

    scale = sel(SCALES, jnp.float32)
    szf = sel(SIZES, jnp.float32)
    szi = sel(SIZES, jnp.int32)
    base = sel(BASES, jnp.int32)

    x1s = x1 * scale
    y1s = y1 * scale
    roi_w = jnp.maximum(x2 * scale - x1s, 1.0)
    roi_h = jnp.maximum(y2 * scale - y1s, 1.0)
    bin_w = roi_w / OUT_W
    bin_h = roi_h / OUT_H

    phf = ph.astype(jnp.float32)
    pwf = pw.astype(jnp.float32)

    ys = []
    xs = []
    for s in range(2):
        off = 0.25 + 0.5 * s
        yv = y1s + (phf + off) * bin_h
        xv = x1s + (pwf + off) * bin_w
        vy = (yv > -1.0) & (yv < szf)
        vx = (xv > -1.0) & (xv < szf)
        yc = jnp.clip(yv, 0.0, szf - 1.0)
        xc = jnp.clip(xv, 0.0, szf - 1.0)
        y0 = jnp.minimum(jnp.floor(yc), szf - 2.0)
        x0 = jnp.minimum(jnp.floor(xc), szf - 2.0)
        ly = yc - y0
        lx = xc - x0
        ys.append((y0.astype(jnp.int32), ly, 1.0 - ly, vy))
        xs.append((x0.astype(jnp.int32), lx, 1.0 - lx, vx))

    for sy in range(2):
        y0i, ly, hy, vy = ys[sy]
        for sx in range(2):
            x0i, lx, hx, vx = xs[sx]
            q = jnp.where(vy & vx, 0.25, 0.0)
            i00 = base + y0i * szi + x0i
            j = (sy * 2 + sx) * 4
            idx_ref[0, j] = i00
            idx_ref[0, j + 1] = i00 + 1
            idx_ref[0, j + 2] = i00 + szi
            idx_ref[0, j + 3] = i00 + szi + 1
            wt_ref[0, j] = hy * hx * q
            wt_ref[0, j + 1] = hy * lx * q
            wt_ref[0, j + 2] = ly * hx * q
            wt_ref[0, j + 3] = ly * lx * q


def _make_sc_gather(kpad):
    rows_per_tile = kpad * NBINS // NW
    nchunk = rows_per_tile // G

    mesh = plsc.VectorSubcoreMesh(core_axis_name="c", subcore_axis_name="s")

    @functools.partial(
        pl.kernel,
        out_type=jax.ShapeDtypeStruct((kpad * NBINS, C), jnp.float32),
        mesh=mesh,
        scratch_types=[
            pltpu.VMEM((nchunk, G * NCON), jnp.int32),
            pltpu.VMEM((nchunk * G * NCON,), jnp.float32),
            pltpu.VMEM((G * NCON, C), jnp.float32),
            pltpu.VMEM((G * NCON, C), jnp.float32),
            pltpu.VMEM((G, C), jnp.float32),
            pltpu.VMEM((G, C), jnp.float32),
            pltpu.SemaphoreType.DMA,
            pltpu.SemaphoreType.DMA,
            pltpu.SemaphoreType.DMA,
            pltpu.SemaphoreType.DMA,
        ],
    )
    def sc_gather(idx_hbm, wt_hbm, table_hbm, out_hbm,
                  idx_v, wt_v, rows0, rows1, ob0, ob1, gs0, gs1, os0, os1):
        wid = lax.axis_index("s") * NC + lax.axis_index("c")
        base_row = wid * rows_per_tile
        pltpu.sync_copy(idx_hbm.at[wid], idx_v)
        pltpu.sync_copy(wt_hbm.at[wid], wt_v)

        rows = (rows0, rows1)
        outs = (ob0, ob1)
        gsems = (gs0, gs1)
        osems = (os0, os1)

        pltpu.async_copy(table_hbm.at[idx_v.at[0]], rows0, gs0)
        pltpu.async_copy(table_hbm.at[idx_v.at[1]], rows1, gs1)

        @pl.loop(0, nchunk, step=2)
        def _chunks(g0):
            for half in range(2):
                g = g0 + half
                rb = rows[half]
                ob = outs[half]
                gsem = gsems[half]
                osem = osems[half]

                pltpu.make_async_copy(table_hbm.at[idx_v.at[g]], rb, gsem).wait()

                @pl.when(g >= 2)
                def _wait_out():
                    pltpu.make_async_copy(
                        ob, out_hbm.at[pl.ds(base_row, G)], osem).wait()

                @pl.loop(0, G)
                def _row(r):
                    rbase = r * NCON
                    wbase = g * (G * NCON) + rbase
                    w16 = wt_v[pl.ds(wbase, 16)]
                    acc = [jnp.zeros((16,), jnp.float32) for _ in range(16)]
                    for j in range(NCON):
                        wj = jnp.full((16,), w16[j], jnp.float32)
                        for cc in range(16):
                            acc[cc] = acc[cc] + wj * rb[rbase + j,
                                                        pl.ds(cc * 16, 16)]
                    for cc in range(16):
                        ob[r, pl.ds(cc * 16, 16)] = acc[cc]

                pltpu.async_copy(
                    ob, out_hbm.at[pl.ds(base_row + g * G, G)], osem)

                @pl.when(g + 2 < nchunk)
                def _next_gather():
                    pltpu.async_copy(table_hbm.at[idx_v.at[g + 2]], rb, gsem)

        pltpu.make_async_copy(ob0, out_hbm.at[pl.ds(base_row, G)], os0).wait()
        pltpu.make_async_copy(ob1, out_hbm.at[pl.ds(base_row, G)], os1).wait()

    return sc_gather


def kernel(feat0, feat1, feat2, feat3, boxes):
    k = boxes.shape[0]
    kpad = ((k + 255) // 256) * 256
    sub = kpad // 128

    area = (boxes[:, 2] - boxes[:, 0]) * (boxes[:, 3] - boxes[:, 1])
    s = jnp.sqrt(area)
    target = jnp.floor(4.0 + jnp.log2(s / 224.0) + 1e-6)
    lvls = jnp.clip(target, 2, 5).astype(jnp.int32) - 2
    lv = jnp.zeros((kpad,), jnp.int32).at[:k].set(lvls).reshape(sub, 128)

    boxes_p = jnp.zeros((kpad, 4), boxes.dtype).at[:k].set(boxes)
    bx = boxes_p.T.reshape(4, sub, 128)

    idx4, wt4 = pl.pallas_call(
        _iw_kernel,
        grid=(NBINS,),
        in_specs=[
            pl.BlockSpec((4, sub, 128), lambda b: (0, 0, 0)),
            pl.BlockSpec((sub, 128), lambda b: (0, 0)),
        ],
        out_specs=[
            pl.BlockSpec((1, NCON, sub, 128), lambda b: (b, 0, 0, 0)),
            pl.BlockSpec((1, NCON, sub, 128), lambda b: (b, 0, 0, 0)),
        ],
        out_shape=[
            jax.ShapeDtypeStruct((NBINS, NCON, sub, 128), jnp.int32),
            jax.ShapeDtypeStruct((NBINS, NCON, sub, 128), jnp.float32),
        ],
    )(bx, lv)

    rows_per_tile = kpad * NBINS // NW
    nchunk = rows_per_tile // G
    idx2 = (idx4.reshape(NBINS, NCON, kpad).transpose(2, 0, 1)
            .reshape(NW, nchunk, G * NCON))
    wt2 = (wt4.reshape(NBINS, NCON, kpad).transpose(2, 0, 1)
           .reshape(NW, nchunk * G * NCON))

    table = jnp.concatenate([
        f[0].transpose(1, 2, 0).reshape(-1, C)
        for f in (feat0, feat1, feat2, feat3)
    ], axis=0)

    out = _make_sc_gather(kpad)(idx2, wt2, table)

    res = (out.reshape(kpad, NBINS, C)[:k].transpose(0, 2, 1)
           .reshape(k, C, OUT_H, OUT_W))
    return res


# speedup vs baseline: 9.2992x; 1.0144x over previous
import functools

import jax
import jax.numpy as jnp
from jax import lax
from jax.experimental import pallas as pl
from jax.experimental.pallas import tpu as pltpu
from jax.experimental.pallas import tpu_sc as plsc

SCALES = (0.25, 0.125, 0.0625, 0.03125)
SIZES = (256, 128, 64, 32)
BASES = (0, 65536, 81920, 86016)
C = 256
OUT_H = 7
OUT_W = 7
NBINS = OUT_H * OUT_W
NCON = 16
NC = 2
NS = 16
NW = NC * NS
G = 8


def _iw_kernel(bx_ref, lv_ref, idx_ref, wt_ref):
    b = pl.program_id(0)
    ph = b // OUT_W
    pw = b - ph * OUT_W
    x1 = bx_ref[0]
    y1 = bx_ref[1]
    x2 = bx_ref[2]
    y2 = bx_ref[3]
    lv = lv_ref[...]

    def sel(vals, dtype):
        out = jnp.full(lv.shape, vals[3], dtype)
        for l in (2, 1, 0):
            out = jnp.where(lv == l, jnp.asarray(vals[l], dtype), out)
        return out

    scale = sel(SCALES, jnp.float32)
    szf = sel(SIZES, jnp.float32)
    szi = sel(SIZES, jnp.int32)
    base = sel(BASES, jnp.int32)

    x1s = x1 * scale
    y1s = y1 * scale
    roi_w = jnp.maximum(x2 * scale - x1s, 1.0)
    roi_h = jnp.maximum(y2 * scale - y1s, 1.0)
    bin_w = roi_w / OUT_W
    bin_h = roi_h / OUT_H

    phf = ph.astype(jnp.float32)
    pwf = pw.astype(jnp.float32)

    ys = []
    xs = []
    for s in range(2):
        off = 0.25 + 0.5 * s
        yv = y1s + (phf + off) * bin_h
        xv = x1s + (pwf + off) * bin_w
        vy = (yv > -1.0) & (yv < szf)
        vx = (xv > -1.0) & (xv < szf)
        yc = jnp.clip(yv, 0.0, szf - 1.0)
        xc = jnp.clip(xv, 0.0, szf - 1.0)
        y0 = jnp.minimum(jnp.floor(yc), szf - 2.0)
        x0 = jnp.minimum(jnp.floor(xc), szf - 2.0)
        ly = yc - y0
        lx = xc - x0
        ys.append((y0.astype(jnp.int32), ly, 1.0 - ly, vy))
        xs.append((x0.astype(jnp.int32), lx, 1.0 - lx, vx))

    for sy in range(2):
        y0i, ly, hy, vy = ys[sy]
        for sx in range(2):
            x0i, lx, hx, vx = xs[sx]
            q = jnp.where(vy & vx, 0.25, 0.0)
            i00 = base + y0i * szi + x0i
            j = (sy * 2 + sx) * 4
            idx_ref[0, j] = i00
            idx_ref[0, j + 1] = i00 + 1
            idx_ref[0, j + 2] = i00 + szi
            idx_ref[0, j + 3] = i00 + szi + 1
            wt_ref[0, j] = hy * hx * q
            wt_ref[0, j + 1] = hy * lx * q
            wt_ref[0, j + 2] = ly * hx * q
            wt_ref[0, j + 3] = ly * lx * q


def _make_sc_gather(kpad):
    rows_per_tile = kpad * NBINS // NW
    nchunk = rows_per_tile // G

    mesh = plsc.VectorSubcoreMesh(core_axis_name="c", subcore_axis_name="s")

    @functools.partial(
        pl.kernel,
        out_type=jax.ShapeDtypeStruct((kpad * NBINS, C), jnp.float32),
        mesh=mesh,
        scratch_types=[
            pltpu.VMEM((nchunk, G * NCON), jnp.int32),
            pltpu.VMEM((nchunk * G * NCON,), jnp.float32),
            pltpu.VMEM((G * NCON, C), jnp.float32),
            pltpu.VMEM((G * NCON, C), jnp.float32),
            pltpu.VMEM((G, C), jnp.float32),
            pltpu.VMEM((G, C), jnp.float32),
            pltpu.SemaphoreType.DMA,
            pltpu.SemaphoreType.DMA,
            pltpu.SemaphoreType.DMA,
            pltpu.SemaphoreType.DMA,
        ],
    )
    def sc_gather(idx_hbm, wt_hbm, table_hbm, out_hbm,
                  idx_v, wt_v, rows0, rows1, ob0, ob1, gs0, gs1, os0, os1):
        wid = lax.axis_index("s") * NC + lax.axis_index("c")
        base_row = wid * rows_per_tile
        pltpu.sync_copy(idx_hbm.at[wid], idx_v)
        pltpu.sync_copy(wt_hbm.at[wid], wt_v)

        rows = (rows0, rows1)
        outs = (ob0, ob1)
        gsems = (gs0, gs1)
        osems = (os0, os1)

        pltpu.async_copy(table_hbm.at[idx_v.at[0]], rows0, gs0)
        pltpu.async_copy(table_hbm.at[idx_v.at[1]], rows1, gs1)

        @pl.loop(0, nchunk, step=2)
        def _chunks(g0):
            for half in range(2):
                g = g0 + half
                rb = rows[half]
                ob = outs[half]
                gsem = gsems[half]
                osem = osems[half]

                pltpu.make_async_copy(table_hbm.at[idx_v.at[g]], rb, gsem).wait()

                @pl.when(g >= 2)
                def _wait_out():
                    pltpu.make_async_copy(
                        ob, out_hbm.at[pl.ds(base_row, G)], osem).wait()

                @pl.loop(0, G)
                def _row(r):
                    rbase = r * NCON
                    wbase = g * (G * NCON) + rbase
                    w16 = wt_v[pl.ds(wbase, 16)]
                    wjs = [jnp.full((16,), w16[j], jnp.float32)
                           for j in range(NCON)]
                    for cb in range(4):
                        accs = [wjs[0] * rb[rbase, pl.ds((cb * 4 + q) * 16, 16)]
                                for q in range(4)]
                        for j in range(1, NCON):
                            for q in range(4):
                                accs[q] = accs[q] + wjs[j] * rb[
                                    rbase + j, pl.ds((cb * 4 + q) * 16, 16)]
                        for q in range(4):
                            ob[r, pl.ds((cb * 4 + q) * 16, 16)] = accs[q]

                pltpu.async_copy(
                    ob, out_hbm.at[pl.ds(base_row + g * G, G)], osem)

                @pl.when(g + 2 < nchunk)
                def _next_gather():
                    pltpu.async_copy(table_hbm.at[idx_v.at[g + 2]], rb, gsem)

        pltpu.make_async_copy(ob0, out_hbm.at[pl.ds(base_row, G)], os0).wait()
        pltpu.make_async_copy(ob1, out_hbm.at[pl.ds(base_row, G)], os1).wait()

    return sc_gather


def kernel(feat0, feat1, feat2, feat3, boxes):
    k = boxes.shape[0]
    kpad = ((k + 255) // 256) * 256
    sub = kpad // 128

    area = (boxes[:, 2] - boxes[:, 0]) * (boxes[:, 3] - boxes[:, 1])
    s = jnp.sqrt(area)
    target = jnp.floor(4.0 + jnp.log2(s / 224.0) + 1e-6)
    lvls = jnp.clip(target, 2, 5).astype(jnp.int32) - 2
    lv = jnp.zeros((kpad,), jnp.int32).at[:k].set(lvls).reshape(sub, 128)

    boxes_p = jnp.zeros((kpad, 4), boxes.dtype).at[:k].set(boxes)
    bx = boxes_p.T.reshape(4, sub, 128)

    idx4, wt4 = pl.pallas_call(
        _iw_kernel,
        grid=(NBINS,),
        in_specs=[
            pl.BlockSpec((4, sub, 128), lambda b: (0, 0, 0)),
            pl.BlockSpec((sub, 128), lambda b: (0, 0)),
        ],
        out_specs=[
            pl.BlockSpec((1, NCON, sub, 128), lambda b: (b, 0, 0, 0)),
            pl.BlockSpec((1, NCON, sub, 128), lambda b: (b, 0, 0, 0)),
        ],
        out_shape=[
            jax.ShapeDtypeStruct((NBINS, NCON, sub, 128), jnp.int32),
            jax.ShapeDtypeStruct((NBINS, NCON, sub, 128), jnp.float32),
        ],
    )(bx, lv)

    rows_per_tile = kpad * NBINS // NW
    nchunk = rows_per_tile // G
    idx2 = (idx4.reshape(NBINS, NCON, kpad).transpose(2, 0, 1)
            .reshape(NW, nchunk, G * NCON))
    wt2 = (wt4.reshape(NBINS, NCON, kpad).transpose(2, 0, 1)
           .reshape(NW, nchunk * G * NCON))

    table = jnp.concatenate([
        f[0].transpose(1, 2, 0).reshape(-1, C)
        for f in (feat0, feat1, feat2, feat3)
    ], axis=0)

    out = _make_sc_gather(kpad)(idx2, wt2, table)

    res = (out.reshape(kpad, NBINS, C)[:k].transpose(0, 2, 1)
           .reshape(k, C, OUT_H, OUT_W))
    return res
